# Initial kernel scaffold; baseline (speedup 1.0000x reference)
import jax
import jax.numpy as jnp
from jax.experimental import pallas as pl


def kernel(x, node_outputs, node_foci):
    raise NotImplementedError("write your pallas kernel here")


# speedup vs baseline: 1556.0745x; 1556.0745x over previous
import functools

import jax
import jax.numpy as jnp
from jax import lax
from jax.experimental import pallas as pl
from jax.experimental.pallas import tpu as pltpu
from jax.experimental.pallas import tpu_sc as plsc

N_TREES = 128
MAX_DEPTH = 10
N_NODES = 2 ** MAX_DEPTH - 1
BATCH = 16384
N_FEAT = 512
N_WORDS = N_FEAT // 32

NC = 2
NS = 16
NW = NC * NS
TP = 4
RP = NW // TP
TREES_PER = N_TREES // TP
ROWS_PER = BATCH // RP
CHUNK = 1024
N_CHUNKS = ROWS_PER // CHUNK


def _make_forest_kernel():
  mesh = plsc.VectorSubcoreMesh(
      core_axis_name="c", subcore_axis_name="s", num_cores=NC,
      num_subcores=NS)

  @functools.partial(
      pl.kernel,
      out_type=jax.ShapeDtypeStruct((N_TREES, BATCH), jnp.int32),
      mesh=mesh,
      scratch_types=[
          pltpu.VMEM((TREES_PER, N_NODES), jnp.int32),
          pltpu.VMEM((CHUNK, N_WORDS), jnp.int32),
          pltpu.VMEM((TREES_PER, CHUNK), jnp.int32),
      ],
      compiler_params=pltpu.CompilerParams(
          use_tc_tiling_on_sc=False, needs_layout_passes=False),
  )
  def forest(xw_hbm, tbl_hbm, out_hbm, tbl_v, xw_v, out_v):
    wid = lax.axis_index("s") * NC + lax.axis_index("c")
    tp = wid // RP
    rp = wid % RP

    pltpu.sync_copy(tbl_hbm.at[pl.ds(tp * TREES_PER, TREES_PER), :], tbl_v)

    lane = lax.iota(jnp.int32, 16)

    for chunk in range(N_CHUNKS):
      row0 = rp * ROWS_PER + chunk * CHUNK
      pltpu.sync_copy(xw_hbm.at[pl.ds(row0, CHUNK), :], xw_v)

      def tree_body(tl, _):
        tlv = jnp.full((16,), tl, dtype=jnp.int32)

        @plsc.parallel_loop(0, CHUNK // 16, unroll=4)
        def rg_body(rg):
          rowv = rg * 16 + lane
          node = jnp.zeros((16,), jnp.int32)
          acc = jnp.zeros((16,), jnp.int32)
          for d in range(MAX_DEPTH):
            tv = plsc.load_gather(tbl_v, [tlv, node])
            f = tv & 511
            w = plsc.load_gather(xw_v, [rowv, f >> 5])
            dec = (w >> (f & 31)) & 1
            acc = acc | (((tv >> (16 + dec)) & 1) << d)
            node = node + node + dec + 1
          out_v[tl, pl.ds(rg * 16, 16)] = acc

        return 0

      lax.fori_loop(0, TREES_PER, tree_body, 0)
      pltpu.sync_copy(
          out_v,
          out_hbm.at[pl.ds(tp * TREES_PER, TREES_PER), pl.ds(row0, CHUNK)])

  return forest


_forest = _make_forest_kernel()


def kernel(x, node_outputs, node_foci):
  xi = x.astype(jnp.int32).reshape(BATCH, N_WORDS, 32)
  shifts = jnp.left_shift(jnp.int32(1), jnp.arange(32, dtype=jnp.int32))
  xw = jnp.sum(xi * shifts, axis=-1, dtype=jnp.int32)

  tbl = (node_foci.astype(jnp.int32)
         | (node_outputs[..., 0].astype(jnp.int32) << 16)
         | (node_outputs[..., 1].astype(jnp.int32) << 17))

  out_words = _forest(xw, tbl)

  depths = jnp.arange(MAX_DEPTH, dtype=jnp.int32)[:, None, None]
  bits = (out_words[None, :, :] >> depths) & 1
  out_bools = jnp.transpose(bits, (2, 0, 1)).reshape(BATCH, MAX_DEPTH * N_TREES) > 0
  x_cat = jnp.concatenate([x, out_bools], axis=1)
  output = out_bools[:, (MAX_DEPTH - 1) * N_TREES:]
  return (x_cat, output)
